# baseline (speedup 1.0000x reference)
import functools

import jax
import jax.numpy as jnp
from jax import lax
from jax.experimental import pallas as pl
from jax.experimental.pallas import tpu as pltpu
from jax.experimental.pallas import tpu_sc as plsc

_N = 2_000_000
_S = 50_000
_SP = 50_048
_C = 3200
_NW = 32
_TOTAL_CHUNKS = _N // _C
_MAX_SLOTS = -(-_TOTAL_CHUNKS // _NW)
_L = 16
_MASK_HI = -65536


def _prep_body(a_ref, u_ref, l_ref, g_ref, p1_ref, p2_ref):
    a_ = jnp.maximum(a_ref[...], 0.0)
    u_ = jnp.maximum(u_ref[...], 0.0)
    lam = 0.2 * jax.nn.sigmoid(l_ref[...])
    lg = lam * jax.nn.sigmoid(g_ref[...])

    def pack(hi, lo):
        hb = lax.bitcast_convert_type(hi, jnp.int32)
        lb = lax.bitcast_convert_type(lo, jnp.int32)
        hb = (hb + 0x8000) & _MASK_HI
        lbr = lax.shift_right_logical(lb + 0x8000, 16)
        return hb | lbr

    p1_ref[...] = pack(a_, u_)
    p2_ref[...] = pack(lam, lg)


_prep = pl.pallas_call(
    _prep_body,
    out_shape=(jax.ShapeDtypeStruct((_S,), jnp.int32),
               jax.ShapeDtypeStruct((_S,), jnp.int32)),
)


def _make_sc_kernel():
    mesh = plsc.VectorSubcoreMesh(core_axis_name="c", subcore_axis_name="s")

    @functools.partial(
        pl.kernel,
        mesh=mesh,
        compiler_params=pltpu.CompilerParams(needs_layout_passes=False),
        out_type=jax.ShapeDtypeStruct((_NW, _L), jnp.float32),
        scratch_types=[
            pltpu.VMEM((_SP,), jnp.int32),
            pltpu.VMEM((_SP,), jnp.int32),
            pltpu.VMEM((_C,), jnp.float32),
            pltpu.VMEM((_C,), jnp.float32),
            pltpu.VMEM((_C,), jnp.float32),
            pltpu.VMEM((_C,), jnp.int32),
            pltpu.VMEM((_C,), jnp.float32),
            pltpu.VMEM((_C,), jnp.float32),
            pltpu.VMEM((_C,), jnp.float32),
            pltpu.VMEM((_C,), jnp.int32),
            pltpu.VMEM((_L,), jnp.float32),
            pltpu.SemaphoreType.DMA,
            pltpu.SemaphoreType.DMA,
            pltpu.SemaphoreType.DMA,
        ],
    )
    def sc_partial(y_hbm, j_hbm, k_hbm, sub_hbm, p1_hbm, p2_hbm,
                   out_hbm, p1_v, p2_v,
                   ya, ja, ka, sa, yb, jb, kb, sb, acc_v, sema, semb, semt):
        cid = lax.axis_index("c")
        sid = lax.axis_index("s")
        wid = sid * 2 + cid

        pltpu.async_copy(p1_hbm, p1_v.at[pl.ds(0, _S)], semt)
        pltpu.async_copy(p2_hbm, p2_v.at[pl.ds(0, _S)], semt)

        def slot_base(s):
            return pl.multiple_of((wid + s * _NW) * _C, 8)

        def issue(s, yv, jv, kv, sv, sem):
            @pl.when(wid + s * _NW < _TOTAL_CHUNKS)
            def _():
                base = slot_base(s)
                pltpu.async_copy(y_hbm.at[pl.ds(base, _C)], yv, sem)
                pltpu.async_copy(j_hbm.at[pl.ds(base, _C)], jv, sem)
                pltpu.async_copy(k_hbm.at[pl.ds(base, _C)], kv, sem)
                pltpu.async_copy(sub_hbm.at[pl.ds(base, _C)], sv, sem)

        def consume(s, yv, jv, kv, sv, sem):
            @pl.when(wid + s * _NW < _TOTAL_CHUNKS)
            def _():
                base = slot_base(s)
                pltpu.make_async_copy(y_hbm.at[pl.ds(base, _C)], yv, sem).wait()
                pltpu.make_async_copy(j_hbm.at[pl.ds(base, _C)], jv, sem).wait()
                pltpu.make_async_copy(k_hbm.at[pl.ds(base, _C)], kv, sem).wait()
                pltpu.make_async_copy(sub_hbm.at[pl.ds(base, _C)], sv, sem).wait()

                def vec_body(t, a2):
                    sl = pl.ds(t * _L, _L)
                    idx = sv[sl]
                    w1 = plsc.load_gather(p1_v, [idx])
                    w2 = plsc.load_gather(p2_v, [idx])
                    a_ = lax.bitcast_convert_type(w1 & _MASK_HI, jnp.float32)
                    u_ = lax.bitcast_convert_type(w1 << 16, jnp.float32)
                    l_ = lax.bitcast_convert_type(w2 & _MASK_HI, jnp.float32)
                    lg = lax.bitcast_convert_type(w2 << 16, jnp.float32)
                    mu = a_ - u_ * jnp.exp(-(l_ * jv[sl] + lg * kv[sl]))
                    d = yv[sl] - mu
                    return a2 + d * d

                contrib = plsc.parallel_loop(
                    0, _C // _L, unroll=32,
                    carry=jnp.zeros((_L,), jnp.float32))(vec_body)
                acc_v[...] = acc_v[...] + contrib

        acc_v[...] = jnp.zeros((_L,), jnp.float32)
        issue(0, ya, ja, ka, sa, sema)
        pltpu.make_async_copy(p1_hbm, p1_v.at[pl.ds(0, _S)], semt).wait()
        pltpu.make_async_copy(p2_hbm, p2_v.at[pl.ds(0, _S)], semt).wait()

        def pair_body(p, carry):
            s0 = 2 * p
            issue(s0 + 1, yb, jb, kb, sb, semb)
            consume(s0, ya, ja, ka, sa, sema)
            issue(s0 + 2, ya, ja, ka, sa, sema)
            consume(s0 + 1, yb, jb, kb, sb, semb)
            return carry

        lax.fori_loop(0, _MAX_SLOTS // 2, pair_body, jnp.int32(0))
        pltpu.sync_copy(acc_v, out_hbm.at[wid])

    return sc_partial


_sc_partial = _make_sc_kernel()


def _finish_body(p_ref, o_ref):
    o_ref[0, 0] = jnp.sqrt(jnp.sum(p_ref[...]) / _N)


_finish = pl.pallas_call(
    _finish_body,
    out_shape=jax.ShapeDtypeStruct((1, 1), jnp.float32),
    out_specs=pl.BlockSpec(memory_space=pltpu.SMEM),
)


def kernel(y, j, k, sub, A, U, Lambda, Gamma1):
    p1, p2 = _prep(A, U, Lambda, Gamma1)
    partials = _sc_partial(y, j, k, sub.astype(jnp.int32), p1, p2)
    return _finish(partials)[0, 0]


# speedup vs baseline: 1.8650x; 1.8650x over previous
import functools

import jax
import jax.numpy as jnp
from jax import lax
from jax.experimental import pallas as pl
from jax.experimental.pallas import tpu as pltpu
from jax.experimental.pallas import tpu_sc as plsc

_N = 2_000_000
_S = 50_000
_SP = 50_048
_C = 3200
_NW = 32
_TOTAL_CHUNKS = _N // _C
_MAX_SLOTS = -(-_TOTAL_CHUNKS // _NW)
_L = 16
_MASK_HI = -65536


def _prep_body(a_ref, u_ref, l_ref, g_ref, p1_ref, p2_ref):
    a_ = jnp.maximum(a_ref[...], 0.0)
    u_ = jnp.maximum(u_ref[...], 0.0)
    lam = 0.2 * jax.nn.sigmoid(l_ref[...])
    lg = lam * jax.nn.sigmoid(g_ref[...])

    def pack(hi, lo):
        hb = lax.bitcast_convert_type(hi, jnp.int32)
        lb = lax.bitcast_convert_type(lo, jnp.int32)
        hb = (hb + 0x8000) & _MASK_HI
        lbr = lax.shift_right_logical(lb + 0x8000, 16)
        return hb | lbr

    p1_ref[...] = pack(a_, u_)
    p2_ref[...] = pack(lam, lg)


_prep = pl.pallas_call(
    _prep_body,
    out_shape=(jax.ShapeDtypeStruct((_S,), jnp.int32),
               jax.ShapeDtypeStruct((_S,), jnp.int32)),
)


def _make_sc_kernel():
    mesh = plsc.VectorSubcoreMesh(core_axis_name="c", subcore_axis_name="s")

    @functools.partial(
        pl.kernel,
        mesh=mesh,
        compiler_params=pltpu.CompilerParams(needs_layout_passes=False),
        out_type=jax.ShapeDtypeStruct((_NW, _L), jnp.float32),
        scratch_types=[
            pltpu.VMEM((_SP,), jnp.int32),
            pltpu.VMEM((_SP,), jnp.int32),
            pltpu.VMEM((_C,), jnp.float32),
            pltpu.VMEM((_C,), jnp.float32),
            pltpu.VMEM((_C,), jnp.float32),
            pltpu.VMEM((_C,), jnp.int32),
            pltpu.VMEM((_C,), jnp.float32),
            pltpu.VMEM((_C,), jnp.float32),
            pltpu.VMEM((_C,), jnp.float32),
            pltpu.VMEM((_C,), jnp.int32),
            pltpu.VMEM((_L,), jnp.float32),
            pltpu.SemaphoreType.DMA,
            pltpu.SemaphoreType.DMA,
            pltpu.SemaphoreType.DMA,
        ],
    )
    def sc_partial(y_hbm, j_hbm, k_hbm, sub_hbm, p1_hbm, p2_hbm,
                   out_hbm, p1_v, p2_v,
                   ya, ja, ka, sa, yb, jb, kb, sb, acc_v, sema, semb, semt):
        cid = lax.axis_index("c")
        sid = lax.axis_index("s")
        wid = sid * 2 + cid

        pltpu.async_copy(p1_hbm, p1_v.at[pl.ds(0, _S)], semt)
        pltpu.async_copy(p2_hbm, p2_v.at[pl.ds(0, _S)], semt)

        def slot_base(s):
            return pl.multiple_of((wid + s * _NW) * _C, 8)

        def issue(s, yv, jv, kv, sv, sem):
            @pl.when(wid + s * _NW < _TOTAL_CHUNKS)
            def _():
                base = slot_base(s)
                pltpu.async_copy(y_hbm.at[pl.ds(base, _C)], yv, sem)
                pltpu.async_copy(j_hbm.at[pl.ds(base, _C)], jv, sem)
                pltpu.async_copy(k_hbm.at[pl.ds(base, _C)], kv, sem)
                pltpu.async_copy(sub_hbm.at[pl.ds(base, _C)], sv, sem)

        def consume(s, yv, jv, kv, sv, sem):
            @pl.when(wid + s * _NW < _TOTAL_CHUNKS)
            def _():
                base = slot_base(s)
                pltpu.make_async_copy(y_hbm.at[pl.ds(base, _C)], yv, sem).wait()
                pltpu.make_async_copy(j_hbm.at[pl.ds(base, _C)], jv, sem).wait()
                pltpu.make_async_copy(k_hbm.at[pl.ds(base, _C)], kv, sem).wait()
                pltpu.make_async_copy(sub_hbm.at[pl.ds(base, _C)], sv, sem).wait()

                def vec_body(t, a2):
                    sl = pl.ds(t * _L, _L)
                    idx = sv[sl]
                    w1 = plsc.load_gather(p1_v, [idx])
                    w2 = plsc.load_gather(p2_v, [idx])
                    a_ = lax.bitcast_convert_type(w1 & _MASK_HI, jnp.float32)
                    u_ = lax.bitcast_convert_type(w1 << 16, jnp.float32)
                    l_ = lax.bitcast_convert_type(w2 & _MASK_HI, jnp.float32)
                    lg = lax.bitcast_convert_type(w2 << 16, jnp.float32)
                    mu = a_ - u_ * jnp.exp(-(l_ * jv[sl] + lg * kv[sl]))
                    d = yv[sl] - mu
                    return a2 + d * d

                contrib = plsc.parallel_loop(
                    0, _C // _L, unroll=8,
                    carry=jnp.zeros((_L,), jnp.float32))(vec_body)
                acc_v[...] = acc_v[...] + contrib

        acc_v[...] = jnp.zeros((_L,), jnp.float32)
        issue(0, ya, ja, ka, sa, sema)
        pltpu.make_async_copy(p1_hbm, p1_v.at[pl.ds(0, _S)], semt).wait()
        pltpu.make_async_copy(p2_hbm, p2_v.at[pl.ds(0, _S)], semt).wait()

        def pair_body(p, carry):
            s0 = 2 * p
            issue(s0 + 1, yb, jb, kb, sb, semb)
            consume(s0, ya, ja, ka, sa, sema)
            issue(s0 + 2, ya, ja, ka, sa, sema)
            consume(s0 + 1, yb, jb, kb, sb, semb)
            return carry

        lax.fori_loop(0, _MAX_SLOTS // 2, pair_body, jnp.int32(0))
        pltpu.sync_copy(acc_v, out_hbm.at[wid])

    return sc_partial


_sc_partial = _make_sc_kernel()


def _finish_body(p_ref, o_ref):
    o_ref[0, 0] = jnp.sqrt(jnp.sum(p_ref[...]) / _N)


_finish = pl.pallas_call(
    _finish_body,
    out_shape=jax.ShapeDtypeStruct((1, 1), jnp.float32),
    out_specs=pl.BlockSpec(memory_space=pltpu.SMEM),
)


def kernel(y, j, k, sub, A, U, Lambda, Gamma1):
    p1, p2 = _prep(A, U, Lambda, Gamma1)
    partials = _sc_partial(y, j, k, sub.astype(jnp.int32), p1, p2)
    return _finish(partials)[0, 0]
